# baseline (speedup 1.0000x reference)
import functools

import jax
import jax.numpy as jnp
from jax.experimental import pallas as pl
from jax.experimental.pallas import tpu as pltpu


def _body(x_ref, emb_ref, fh_ref, fw_ref, phh_ref, phw_ref, w1_ref, w2_ref,
          pb_ref, rw_ref, o_ref, ph_ref, pw_ref,
          *, R, H, W, D):
    b = pl.program_id(0)
    i = pl.program_id(1)

    @pl.when(jnp.logical_and(b == 0, i == 0))
    def _init():
        def softplus(v):
            return jnp.maximum(v, 0.0) + jnp.log1p(jnp.exp(-jnp.abs(v)))

        h_pos = jax.lax.broadcasted_iota(jnp.int32, (H, 1), 0).astype(
            jnp.float32) / H
        w_pos = jax.lax.broadcasted_iota(jnp.int32, (W, 1), 0).astype(
            jnp.float32) / W
        fh = softplus(fh_ref[...]) * 10.0
        fw = softplus(fw_ref[...]) * 10.0
        h_ang = h_pos * fh + phh_ref[...]
        w_ang = w_pos * fw + phw_ref[...]
        h_enc = jnp.concatenate([jnp.sin(h_ang), jnp.cos(h_ang)], axis=1)
        w_enc = jnp.concatenate([jnp.sin(w_ang), jnp.cos(w_ang)], axis=1)
        ph_ref[...] = (jnp.dot(h_enc, w1_ref[...],
                               preferred_element_type=jnp.float32)
                       + pb_ref[...])
        pw_ref[...] = jnp.dot(w_enc, w2_ref[...],
                              preferred_element_type=jnp.float32)

    xb = x_ref[0]
    idxf = jnp.floor(jnp.clip(xb * 255.0, 0.0, 255.0))
    iota_v = jax.lax.broadcasted_iota(jnp.int32, (1, 1, 256), 2).astype(
        jnp.float32)
    oh = jnp.where(idxf[:, :, None] == iota_v, 1.0, 0.0)
    oh2 = oh.reshape(R * W, 256).astype(jnp.bfloat16)
    emb = jnp.dot(oh2, emb_ref[...].astype(jnp.bfloat16),
                  preferred_element_type=jnp.float32)
    e = (emb.reshape(R, W, D)
         + ph_ref[pl.ds(i * R, R), :][:, None, :]
         + pw_ref[...][None, :, :])
    ms = jnp.mean(e * e, axis=2, keepdims=True)
    o_ref[0] = e * jax.lax.rsqrt(ms + 1e-6) * rw_ref[...][None, :, :]


def kernel(x, pixel_embed, freq_h, freq_w, phase_h, phase_w, pos_W, pos_b,
           rms_w):
    B, H, W = x.shape
    V, D = pixel_embed.shape
    F = freq_h.shape[0]
    R = 32
    grid = (B, H // R)
    rep = lambda b, i: (0, 0)
    out = pl.pallas_call(
        functools.partial(_body, R=R, H=H, W=W, D=D),
        grid=grid,
        in_specs=[
            pl.BlockSpec((1, R, W), lambda b, i: (b, i, 0)),
            pl.BlockSpec((V, D), rep),
            pl.BlockSpec((1, F), rep),
            pl.BlockSpec((1, F), rep),
            pl.BlockSpec((1, F), rep),
            pl.BlockSpec((1, F), rep),
            pl.BlockSpec((2 * F, D), rep),
            pl.BlockSpec((2 * F, D), rep),
            pl.BlockSpec((1, D), rep),
            pl.BlockSpec((1, D), rep),
        ],
        out_specs=pl.BlockSpec((1, R, W, D), lambda b, i: (b, i, 0, 0)),
        out_shape=jax.ShapeDtypeStruct((B, H, W, D), jnp.float32),
        scratch_shapes=[
            pltpu.VMEM((H, D), jnp.float32),
            pltpu.VMEM((W, D), jnp.float32),
        ],
    )(x, pixel_embed, freq_h.reshape(1, F), freq_w.reshape(1, F),
      phase_h.reshape(1, F), phase_w.reshape(1, F),
      pos_W[:2 * F], pos_W[2 * F:], pos_b.reshape(1, D), rms_w.reshape(1, D))
    return out.reshape(B, H * W, D)


# speedup vs baseline: 1.0336x; 1.0336x over previous
import functools

import jax
import jax.numpy as jnp
from jax.experimental import pallas as pl
from jax.experimental.pallas import tpu as pltpu


def _body(x_ref, emb_ref, fh_ref, fw_ref, phh_ref, phw_ref, w1_ref, w2_ref,
          pb_ref, rw_ref, o_ref, ph_ref, pw_ref,
          *, R, H, W, D):
    b = pl.program_id(0)
    i = pl.program_id(1)

    @pl.when(jnp.logical_and(b == 0, i == 0))
    def _init():
        def softplus(v):
            return jnp.maximum(v, 0.0) + jnp.log1p(jnp.exp(-jnp.abs(v)))

        h_pos = jax.lax.broadcasted_iota(jnp.int32, (H, 1), 0).astype(
            jnp.float32) / H
        w_pos = jax.lax.broadcasted_iota(jnp.int32, (W, 1), 0).astype(
            jnp.float32) / W
        fh = softplus(fh_ref[...]) * 10.0
        fw = softplus(fw_ref[...]) * 10.0
        h_ang = h_pos * fh + phh_ref[...]
        w_ang = w_pos * fw + phw_ref[...]
        h_enc = jnp.concatenate([jnp.sin(h_ang), jnp.cos(h_ang)], axis=1)
        w_enc = jnp.concatenate([jnp.sin(w_ang), jnp.cos(w_ang)], axis=1)
        ph_ref[...] = (jnp.dot(h_enc, w1_ref[...],
                               preferred_element_type=jnp.float32)
                       + pb_ref[...])
        pw_ref[...] = jnp.dot(w_enc, w2_ref[...],
                              preferred_element_type=jnp.float32)

    xb = x_ref[0]
    idxf = jnp.floor(jnp.clip(xb * 255.0, 0.0, 255.0))
    iota_v = jax.lax.broadcasted_iota(jnp.int32, (1, 1, 256), 2).astype(
        jnp.float32)
    oh = jnp.where(idxf[:, :, None] == iota_v, 1.0, 0.0)
    oh2 = oh.reshape(R * W, 256)
    emb = jnp.dot(oh2, emb_ref[...], preferred_element_type=jnp.float32)
    e = (emb.reshape(R, W, D)
         + ph_ref[pl.ds(i * R, R), :][:, None, :]
         + pw_ref[...][None, :, :])
    ms = jnp.mean(e * e, axis=2, keepdims=True)
    o_ref[0] = e * jax.lax.rsqrt(ms + 1e-6) * rw_ref[...][None, :, :]


def kernel(x, pixel_embed, freq_h, freq_w, phase_h, phase_w, pos_W, pos_b,
           rms_w):
    B, H, W = x.shape
    V, D = pixel_embed.shape
    F = freq_h.shape[0]
    R = 64
    grid = (B, H // R)
    rep = lambda b, i: (0, 0)
    out = pl.pallas_call(
        functools.partial(_body, R=R, H=H, W=W, D=D),
        grid=grid,
        in_specs=[
            pl.BlockSpec((1, R, W), lambda b, i: (b, i, 0)),
            pl.BlockSpec((V, D), rep),
            pl.BlockSpec((1, F), rep),
            pl.BlockSpec((1, F), rep),
            pl.BlockSpec((1, F), rep),
            pl.BlockSpec((1, F), rep),
            pl.BlockSpec((2 * F, D), rep),
            pl.BlockSpec((2 * F, D), rep),
            pl.BlockSpec((1, D), rep),
            pl.BlockSpec((1, D), rep),
        ],
        out_specs=pl.BlockSpec((1, R, W, D), lambda b, i: (b, i, 0, 0)),
        out_shape=jax.ShapeDtypeStruct((B, H, W, D), jnp.float32),
        scratch_shapes=[
            pltpu.VMEM((H, D), jnp.float32),
            pltpu.VMEM((W, D), jnp.float32),
        ],
    )(x, pixel_embed, freq_h.reshape(1, F), freq_w.reshape(1, F),
      phase_h.reshape(1, F), phase_w.reshape(1, F),
      pos_W[:2 * F], pos_W[2 * F:], pos_b.reshape(1, D), rms_w.reshape(1, D))
    return out.reshape(B, H * W, D)
